# baseline (speedup 1.0000x reference)
import functools

import jax
import jax.numpy as jnp
from jax import lax
from jax.experimental import pallas as pl
from jax.experimental.pallas import tpu as pltpu
from jax.experimental.pallas import tpu_sc as plsc

B, T, N, E = 2, 4, 10000, 160000
D = 128
DH = D // 2
S = B * T
Np = 10240
NB = 1024
NTILES = 16
TILE_E = E // NTILES
K = 80
NCH = TILE_E // K
NR = 4
S_PER_CORE = S // 2
RPT = Np // NTILES
CSH = 14
CMSK = (1 << CSH) - 1

f32 = jnp.float32
i32 = jnp.int32


def _gelu(v):
    return 0.5 * v * (1.0 + lax.erf(v * 0.7071067811865476))


def _prep_body(x_ref, w_ref, asv_ref, adv_ref, h_ref, as_ref, ad_ref):
    xb = x_ref[0]
    w = w_ref[...]
    h = jnp.dot(xb, w.T, preferred_element_type=f32)
    a_s = jnp.dot(h, asv_ref[0], preferred_element_type=f32)
    a_d = jnp.dot(h, adv_ref[0], preferred_element_type=f32)
    h_ref[0] = h
    as_ref[0] = a_s.reshape(8, 128)
    ad_ref[0] = a_d.reshape(8, 128)


def _prep(xs, W, asv, adv):
    return pl.pallas_call(
        _prep_body,
        grid=(S, Np // NB),
        in_specs=[
            pl.BlockSpec((1, NB, D), lambda s, n: (s, n, 0)),
            pl.BlockSpec((D, D), lambda s, n: (0, 0)),
            pl.BlockSpec((1, D), lambda s, n: (0, 0)),
            pl.BlockSpec((1, D), lambda s, n: (0, 0)),
        ],
        out_specs=[
            pl.BlockSpec((1, NB, D), lambda s, n: (s, n, 0)),
            pl.BlockSpec((1, 8, 128), lambda s, n: (s, n, 0)),
            pl.BlockSpec((1, 8, 128), lambda s, n: (s, n, 0)),
        ],
        out_shape=[
            jax.ShapeDtypeStruct((S, Np, D), f32),
            jax.ShapeDtypeStruct((S, Np // 128, 128), f32),
            jax.ShapeDtypeStruct((S, Np // 128, 128), f32),
        ],
    )(xs, W, asv, adv)


def _mid_body(acc0_ref, acc1_ref, dnp_ref, h_ref, b_ref, w_ref, asv_ref,
              adv_ref, h2_ref, as_ref, ad_ref):
    dn = 1.0 + dnp_ref[0].reshape(NB)
    acc = jnp.concatenate([acc0_ref[0], acc1_ref[0]], axis=-1)
    g = (acc + h_ref[0]) / (dn[:, None] + 1e-16) + b_ref[0]
    z = _gelu(g)
    w = w_ref[...]
    h2 = jnp.dot(z, w.T, preferred_element_type=f32)
    a_s = jnp.dot(h2, asv_ref[0], preferred_element_type=f32)
    a_d = jnp.dot(h2, adv_ref[0], preferred_element_type=f32)
    h2_ref[0] = h2
    as_ref[0] = a_s.reshape(8, 128)
    ad_ref[0] = a_d.reshape(8, 128)


def _mid(acc0, acc1, dnp, h, b, W, asv, adv):
    return pl.pallas_call(
        _mid_body,
        grid=(S, Np // NB),
        in_specs=[
            pl.BlockSpec((1, NB, DH), lambda s, n: (s, n, 0)),
            pl.BlockSpec((1, NB, DH), lambda s, n: (s, n, 0)),
            pl.BlockSpec((1, 8, 128), lambda s, n: (s, n, 0)),
            pl.BlockSpec((1, NB, D), lambda s, n: (s, n, 0)),
            pl.BlockSpec((1, D), lambda s, n: (0, 0)),
            pl.BlockSpec((D, D), lambda s, n: (0, 0)),
            pl.BlockSpec((1, D), lambda s, n: (0, 0)),
            pl.BlockSpec((1, D), lambda s, n: (0, 0)),
        ],
        out_specs=[
            pl.BlockSpec((1, NB, D), lambda s, n: (s, n, 0)),
            pl.BlockSpec((1, 8, 128), lambda s, n: (s, n, 0)),
            pl.BlockSpec((1, 8, 128), lambda s, n: (s, n, 0)),
        ],
        out_shape=[
            jax.ShapeDtypeStruct((S, Np, D), f32),
            jax.ShapeDtypeStruct((S, Np // 128, 128), f32),
            jax.ShapeDtypeStruct((S, Np // 128, 128), f32),
        ],
    )(acc0, acc1, dnp, h, b, W, asv, adv)


def _final_body(acc0_ref, acc1_ref, dnp_ref, h_ref, b_ref, vsi_ref, wt_ref,
                bt_ref, out_ref):
    bi = pl.program_id(0)
    dn = 1.0 + dnp_ref[0].reshape(T, NB)
    acc = jnp.concatenate([acc0_ref[0], acc1_ref[0]], axis=-1)
    o = (acc + h_ref[0]) / (dn[:, :, None] + 1e-16) + b_ref[0]
    v = vsi_ref[bi]
    tmask = (lax.broadcasted_iota(i32, (T, 1, 1), 0) < v).astype(f32)
    fused = o * tmask
    outs = []
    for t in range(T):
        c = fused[0] * wt_ref[t, 0]
        for s in range(1, T):
            c = c + fused[s] * wt_ref[t, s]
        c = c + bt_ref[t]
        outs.append(_gelu(c) + fused[t])
    out_ref[0] = jnp.stack(outs, axis=0)


def _final(acc0, acc1, dnp, h, b, vsi, Wt, bt):
    return pl.pallas_call(
        _final_body,
        grid=(B, Np // NB),
        in_specs=[
            pl.BlockSpec((1, T, NB, DH), lambda bi, n: (bi, 0, n, 0)),
            pl.BlockSpec((1, T, NB, DH), lambda bi, n: (bi, 0, n, 0)),
            pl.BlockSpec((1, T, 8, 128), lambda bi, n: (bi, 0, n, 0)),
            pl.BlockSpec((1, T, NB, D), lambda bi, n: (bi, 0, n, 0)),
            pl.BlockSpec((1, D), lambda bi, n: (0, 0)),
            pl.BlockSpec(memory_space=pltpu.SMEM),
            pl.BlockSpec(memory_space=pltpu.SMEM),
            pl.BlockSpec(memory_space=pltpu.SMEM),
        ],
        out_specs=pl.BlockSpec((1, T, NB, D), lambda bi, n: (bi, 0, n, 0)),
        out_shape=jax.ShapeDtypeStruct((B, T, N, D), f32),
    )(acc0, acc1, dnp, h, b, vsi, Wt, bt)


def _edge_body(hh_hbm, asv_hbm, adv_hbm, zro_hbm, code_hbm,
               acc0_hbm, acc1_hbm, dn_hbm,
               as_t, ad_t, code2d, ex2d, dst_r, fidx_r, exd_r,
               rows0, rows1, rows2, rows3, acc_sh, dn_sh,
               gs0, gs1, gs2, gs3, ss0, ss1, ss2, ss3,
               ds0, ds1, ds2, ds3):
    c = lax.axis_index("c")
    tid = lax.axis_index("s")
    zv = jnp.zeros((16,), f32)
    rows = (rows0, rows1, rows2, rows3)
    gsems = (gs0, gs1, gs2, gs3)
    ssems = (ss0, ss1, ss2, ss3)
    dsems = (ds0, ds1, ds2, ds3)
    accs = (acc0_hbm, acc1_hbm)

    def _snapshot(si, carry):
        s = c * S_PER_CORE + si
        pltpu.sync_copy(asv_hbm.at[s], as_t)
        pltpu.sync_copy(adv_hbm.at[s], ad_t)
        pltpu.sync_copy(code_hbm.at[s, tid], code2d)

        for pp in range(2):

            def _wait_sc(b):
                pltpu.make_async_copy(rows[b], acc_sh.at[dst_r.at[b]],
                                      ssems[b]).wait()
                if pp == 0:
                    pltpu.make_async_copy(exd_r.at[b], dn_sh.at[dst_r.at[b]],
                                          dsems[b]).wait()

            def _prep(ci, b):
                for u in range(K // 16):
                    sl = pl.ds(u * 16, 16)
                    cd = code2d[ci, sl]
                    sv = lax.shift_right_logical(cd, CSH)
                    dv = cd & CMSK
                    if pp == 0:
                        a1 = plsc.load_gather(as_t, [sv])
                        a2 = plsc.load_gather(as_t, [dv])
                        b2 = plsc.load_gather(ad_t, [dv])
                        e = a1 + b2
                        e = jnp.where(e > 0, e, e * 0.2)
                        m = a2 + b2
                        m = jnp.where(m > 0, m, m * 0.2)
                        ex2d[ci, sl] = jnp.exp(e - m)
                        exd_r[b, sl] = ex2d[ci, sl]
                    dst_r[b, sl] = dv
                    fidx_r[b, sl] = (sv + jnp.full((16,), s * Np, i32)) * 2 + pp
                pltpu.async_copy(hh_hbm.at[fidx_r.at[b]], rows[b], gsems[b])

            def _scale_scatter(ci, b):
                pltpu.make_async_copy(hh_hbm.at[fidx_r.at[b]], rows[b],
                                      gsems[b]).wait()

                def _sc16(q, carry2):
                    for r in range(16):
                        j = q * 16 + r
                        w = plsc.load_gather(
                            ex2d, [jnp.full((16,), ci, i32),
                                   jnp.full((16,), j, i32)])
                        for cc in range(4):
                            sl = pl.ds(cc * 16, 16)
                            rows[b][j, sl] = rows[b][j, sl] * w
                    return carry2

                lax.fori_loop(0, K // 16, _sc16, 0)
                pltpu.async_copy(rows[b], acc_sh.at[dst_r.at[b]], ssems[b],
                                 add=True)
                if pp == 0:
                    pltpu.async_copy(exd_r.at[b], dn_sh.at[dst_r.at[b]],
                                     dsems[b], add=True)

            def _zrows(j, carry2):
                for cc in range(4):
                    sl = pl.ds(cc * 16, 16)
                    rows0[j, sl] = zv
                return carry2

            lax.fori_loop(0, K, _zrows, 0)

            def _zacc(z, carry2):
                pltpu.sync_copy(rows0, acc_sh.at[pl.ds(tid * RPT + z * K, K)])
                return carry2

            lax.fori_loop(0, RPT // K, _zacc, 0)
            if pp == 0:
                @pl.when(tid == 0)
                def _zdn():
                    pltpu.sync_copy(zro_hbm, dn_sh)
            plsc.subcore_barrier()

            for i in range(4):
                _prep(i, i)
                if i >= 2:
                    _scale_scatter(i - 2, i - 2)

            def _quad(q, carry2):
                for r in range(4):
                    step = 4 + 4 * q + r
                    _wait_sc(r)
                    _prep(step, r)
                    _scale_scatter(step - 2, (r + 2) % 4)
                return carry2

            lax.fori_loop(0, (NCH - 5) // 4, _quad, 0)
            _wait_sc(0)
            _prep(NCH - 1, 0)
            _scale_scatter(NCH - 3, 2)
            _scale_scatter(NCH - 2, 3)
            _scale_scatter(NCH - 1, 0)
            for b in (1, 2, 3, 0):
                _wait_sc(b)
            plsc.subcore_barrier()

            pltpu.sync_copy(acc_sh.at[pl.ds(tid * RPT, RPT)],
                            accs[pp].at[s, pl.ds(tid * RPT, RPT)])
            if pp == 0:
                pltpu.sync_copy(dn_sh.at[pl.ds(tid * RPT, RPT)],
                                dn_hbm.at[s, pl.ds(tid * RPT, RPT)])
            plsc.subcore_barrier()
        return carry

    lax.fori_loop(0, S_PER_CORE, _snapshot, 0)


def _edge(h_half, asv, adv, zro, code):
    mesh = plsc.VectorSubcoreMesh(core_axis_name="c", subcore_axis_name="s")
    fn = functools.partial(
        pl.kernel,
        mesh=mesh,
        compiler_params=pltpu.CompilerParams(needs_layout_passes=False,
                                             use_tc_tiling_on_sc=False),
        out_type=[
            jax.ShapeDtypeStruct((S, Np, DH), f32),
            jax.ShapeDtypeStruct((S, Np, DH), f32),
            jax.ShapeDtypeStruct((S, Np), f32),
        ],
        scratch_types=[
            pltpu.VMEM((Np,), f32),
            pltpu.VMEM((Np,), f32),
            pltpu.VMEM((NCH, K), i32),
            pltpu.VMEM((NCH, K), f32),
            pltpu.VMEM((NR, K), i32),
            pltpu.VMEM((NR, K), i32),
            pltpu.VMEM((NR, K), f32),
            pltpu.VMEM((K, DH), f32),
            pltpu.VMEM((K, DH), f32),
            pltpu.VMEM((K, DH), f32),
            pltpu.VMEM((K, DH), f32),
            pltpu.VMEM_SHARED((Np, DH), f32),
            pltpu.VMEM_SHARED((Np,), f32),
            pltpu.SemaphoreType.DMA,
            pltpu.SemaphoreType.DMA,
            pltpu.SemaphoreType.DMA,
            pltpu.SemaphoreType.DMA,
            pltpu.SemaphoreType.DMA,
            pltpu.SemaphoreType.DMA,
            pltpu.SemaphoreType.DMA,
            pltpu.SemaphoreType.DMA,
            pltpu.SemaphoreType.DMA,
            pltpu.SemaphoreType.DMA,
            pltpu.SemaphoreType.DMA,
            pltpu.SemaphoreType.DMA,
        ],
    )(_edge_body)
    return fn(h_half, asv, adv, zro, code)


def kernel(x, edge_index, valid_step_index, W1, a_src1, a_dst1, b1,
           W2, a_src2, a_dst2, b2, Wt, bt):
    x = x.astype(f32)
    ei = edge_index.astype(i32)
    code = (lax.shift_left(ei[:, :, 0, :], CSH) | ei[:, :, 1, :]).reshape(
        S, NTILES, NCH, K)
    zro = jnp.zeros((Np,), f32)
    xs = x.reshape(S, N, D)

    h1, as1, ad1 = _prep(xs, W1, a_src1.reshape(1, D), a_dst1.reshape(1, D))
    a1h0, a1h1, dn1 = _edge(h1.reshape(S * Np * 2, DH), as1.reshape(S, Np),
                            ad1.reshape(S, Np), zro, code)
    h2, as2, ad2 = _mid(a1h0, a1h1, dn1.reshape(S, Np // 128, 128), h1,
                        b1.reshape(1, D), W2,
                        a_src2.reshape(1, D), a_dst2.reshape(1, D))
    a2h0, a2h1, dn2 = _edge(h2.reshape(S * Np * 2, DH), as2.reshape(S, Np),
                            ad2.reshape(S, Np), zro, code)
    return _final(a2h0.reshape(B, T, Np, DH), a2h1.reshape(B, T, Np, DH),
                  dn2.reshape(B, T, Np // 128, 128),
                  h2.reshape(B, T, Np, D), b2.reshape(1, D),
                  valid_step_index.astype(i32), Wt.astype(f32), bt.astype(f32))


# speedup vs baseline: 1.3320x; 1.3320x over previous
import functools

import jax
import jax.numpy as jnp
from jax import lax
from jax.experimental import pallas as pl
from jax.experimental.pallas import tpu as pltpu
from jax.experimental.pallas import tpu_sc as plsc

B, T, N, E = 2, 4, 10000, 160000
D = 128
DH = D // 2
S = B * T
Np = 10240
NB = 1024
NTILES = 16
TILE_E = E // NTILES
K = 80
NCH = TILE_E // K
NR = 4
S_PER_CORE = S // 2
RPT = Np // NTILES
CSH = 14
CMSK = (1 << CSH) - 1

f32 = jnp.float32
i32 = jnp.int32


def _gelu(v):
    return 0.5 * v * (1.0 + lax.erf(v * 0.7071067811865476))


def _prep_body(x_ref, w_ref, asv_ref, adv_ref, h_ref, as_ref, ad_ref):
    xb = x_ref[0]
    w = w_ref[...]
    h = jnp.dot(xb, w.T, preferred_element_type=f32)
    a_s = jnp.dot(h, asv_ref[0], preferred_element_type=f32)
    a_d = jnp.dot(h, adv_ref[0], preferred_element_type=f32)
    h_ref[0] = h
    as_ref[0] = a_s.reshape(8, 128)
    ad_ref[0] = a_d.reshape(8, 128)


def _prep(xs, W, asv, adv):
    return pl.pallas_call(
        _prep_body,
        grid=(S, Np // NB),
        in_specs=[
            pl.BlockSpec((1, NB, D), lambda s, n: (s, n, 0)),
            pl.BlockSpec((D, D), lambda s, n: (0, 0)),
            pl.BlockSpec((1, D), lambda s, n: (0, 0)),
            pl.BlockSpec((1, D), lambda s, n: (0, 0)),
        ],
        out_specs=[
            pl.BlockSpec((1, NB, D), lambda s, n: (s, n, 0)),
            pl.BlockSpec((1, 8, 128), lambda s, n: (s, n, 0)),
            pl.BlockSpec((1, 8, 128), lambda s, n: (s, n, 0)),
        ],
        out_shape=[
            jax.ShapeDtypeStruct((S, Np, D), f32),
            jax.ShapeDtypeStruct((S, Np // 128, 128), f32),
            jax.ShapeDtypeStruct((S, Np // 128, 128), f32),
        ],
    )(xs, W, asv, adv)


def _mid_body(acc0_ref, acc1_ref, dnp_ref, h_ref, b_ref, w_ref, asv_ref,
              adv_ref, h2_ref, as_ref, ad_ref):
    dn = 1.0 + dnp_ref[0].reshape(NB)
    acc = jnp.concatenate([acc0_ref[0], acc1_ref[0]], axis=-1)
    g = (acc + h_ref[0]) / (dn[:, None] + 1e-16) + b_ref[0]
    z = _gelu(g)
    w = w_ref[...]
    h2 = jnp.dot(z, w.T, preferred_element_type=f32)
    a_s = jnp.dot(h2, asv_ref[0], preferred_element_type=f32)
    a_d = jnp.dot(h2, adv_ref[0], preferred_element_type=f32)
    h2_ref[0] = h2
    as_ref[0] = a_s.reshape(8, 128)
    ad_ref[0] = a_d.reshape(8, 128)


def _mid(acc0, acc1, dnp, h, b, W, asv, adv):
    return pl.pallas_call(
        _mid_body,
        grid=(S, Np // NB),
        in_specs=[
            pl.BlockSpec((1, NB, DH), lambda s, n: (s, n, 0)),
            pl.BlockSpec((1, NB, DH), lambda s, n: (s, n, 0)),
            pl.BlockSpec((1, 8, 128), lambda s, n: (s, n, 0)),
            pl.BlockSpec((1, NB, D), lambda s, n: (s, n, 0)),
            pl.BlockSpec((1, D), lambda s, n: (0, 0)),
            pl.BlockSpec((D, D), lambda s, n: (0, 0)),
            pl.BlockSpec((1, D), lambda s, n: (0, 0)),
            pl.BlockSpec((1, D), lambda s, n: (0, 0)),
        ],
        out_specs=[
            pl.BlockSpec((1, NB, D), lambda s, n: (s, n, 0)),
            pl.BlockSpec((1, 8, 128), lambda s, n: (s, n, 0)),
            pl.BlockSpec((1, 8, 128), lambda s, n: (s, n, 0)),
        ],
        out_shape=[
            jax.ShapeDtypeStruct((S, Np, D), f32),
            jax.ShapeDtypeStruct((S, Np // 128, 128), f32),
            jax.ShapeDtypeStruct((S, Np // 128, 128), f32),
        ],
    )(acc0, acc1, dnp, h, b, W, asv, adv)


def _final_body(acc0_ref, acc1_ref, dnp_ref, h_ref, b_ref, vsi_ref, wt_ref,
                bt_ref, out_ref):
    bi = pl.program_id(0)
    dn = 1.0 + dnp_ref[0].reshape(T, NB)
    acc = jnp.concatenate([acc0_ref[0], acc1_ref[0]], axis=-1)
    o = (acc + h_ref[0]) / (dn[:, :, None] + 1e-16) + b_ref[0]
    v = vsi_ref[bi]
    tmask = (lax.broadcasted_iota(i32, (T, 1, 1), 0) < v).astype(f32)
    fused = o * tmask
    outs = []
    for t in range(T):
        c = fused[0] * wt_ref[t, 0]
        for s in range(1, T):
            c = c + fused[s] * wt_ref[t, s]
        c = c + bt_ref[t]
        outs.append(_gelu(c) + fused[t])
    out_ref[0] = jnp.stack(outs, axis=0)


def _final(acc0, acc1, dnp, h, b, vsi, Wt, bt):
    return pl.pallas_call(
        _final_body,
        grid=(B, Np // NB),
        in_specs=[
            pl.BlockSpec((1, T, NB, DH), lambda bi, n: (bi, 0, n, 0)),
            pl.BlockSpec((1, T, NB, DH), lambda bi, n: (bi, 0, n, 0)),
            pl.BlockSpec((1, T, 8, 128), lambda bi, n: (bi, 0, n, 0)),
            pl.BlockSpec((1, T, NB, D), lambda bi, n: (bi, 0, n, 0)),
            pl.BlockSpec((1, D), lambda bi, n: (0, 0)),
            pl.BlockSpec(memory_space=pltpu.SMEM),
            pl.BlockSpec(memory_space=pltpu.SMEM),
            pl.BlockSpec(memory_space=pltpu.SMEM),
        ],
        out_specs=pl.BlockSpec((1, T, NB, D), lambda bi, n: (bi, 0, n, 0)),
        out_shape=jax.ShapeDtypeStruct((B, T, N, D), f32),
    )(acc0, acc1, dnp, h, b, vsi, Wt, bt)


def _edge_body(hh_hbm, asv_hbm, adv_hbm, zro_hbm, code_hbm,
               acc0_hbm, acc1_hbm, dn_hbm,
               as_t, ad_t, code2d, ex2d, dst_r, fidx_r, exd_r,
               rows0, rows1, rows2, rows3, acc_sh, dn_sh,
               gs0, gs1, gs2, gs3, ss0, ss1, ss2, ss3,
               ds0, ds1, ds2, ds3):
    c = lax.axis_index("c")
    tid = lax.axis_index("s")
    zv = jnp.zeros((16,), f32)
    rows = (rows0, rows1, rows2, rows3)
    gsems = (gs0, gs1, gs2, gs3)
    ssems = (ss0, ss1, ss2, ss3)
    dsems = (ds0, ds1, ds2, ds3)
    accs = (acc0_hbm, acc1_hbm)

    def _snapshot(si, carry):
        s = c * S_PER_CORE + si
        pltpu.sync_copy(asv_hbm.at[s], as_t)
        pltpu.sync_copy(adv_hbm.at[s], ad_t)
        pltpu.sync_copy(code_hbm.at[s, tid], code2d)

        for pp in range(2):

            def _wait_sc(b):
                pltpu.make_async_copy(rows[b], acc_sh.at[dst_r.at[b]],
                                      ssems[b]).wait()
                if pp == 0:
                    pltpu.make_async_copy(exd_r.at[b], dn_sh.at[dst_r.at[b]],
                                          dsems[b]).wait()

            def _prep(ci, b):
                for u in range(K // 16):
                    sl = pl.ds(u * 16, 16)
                    cd = code2d[ci, sl]
                    sv = lax.shift_right_logical(cd, CSH)
                    dv = cd & CMSK
                    if pp == 0:
                        a1 = plsc.load_gather(as_t, [sv])
                        a2 = plsc.load_gather(as_t, [dv])
                        b2 = plsc.load_gather(ad_t, [dv])
                        e = a1 + b2
                        e = jnp.where(e > 0, e, e * 0.2)
                        m = a2 + b2
                        m = jnp.where(m > 0, m, m * 0.2)
                        ex2d[ci, sl] = jnp.exp(e - m)
                        exd_r[b, sl] = ex2d[ci, sl]
                    dst_r[b, sl] = dv
                    fidx_r[b, sl] = (sv + jnp.full((16,), s * Np, i32)) * 2 + pp
                pltpu.async_copy(hh_hbm.at[fidx_r.at[b]], rows[b], gsems[b])

            def _scale_scatter(ci, b):
                pltpu.make_async_copy(hh_hbm.at[fidx_r.at[b]], rows[b],
                                      gsems[b]).wait()

                def _sc16(q, carry2):
                    for r in range(8):
                        j = q * 8 + r
                        w = plsc.load_gather(
                            ex2d, [jnp.full((16,), ci, i32),
                                   jnp.full((16,), j, i32)])
                        for cc in range(4):
                            sl = pl.ds(cc * 16, 16)
                            rows[b][j, sl] = rows[b][j, sl] * w
                    return carry2

                lax.fori_loop(0, K // 8, _sc16, 0)
                pltpu.async_copy(rows[b], acc_sh.at[dst_r.at[b]], ssems[b],
                                 add=True)
                if pp == 0:
                    pltpu.async_copy(exd_r.at[b], dn_sh.at[dst_r.at[b]],
                                     dsems[b], add=True)

            def _zrows(j, carry2):
                for cc in range(4):
                    sl = pl.ds(cc * 16, 16)
                    rows0[j, sl] = zv
                return carry2

            lax.fori_loop(0, K, _zrows, 0)

            def _zacc(z, carry2):
                pltpu.sync_copy(rows0, acc_sh.at[pl.ds(tid * RPT + z * K, K)])
                return carry2

            lax.fori_loop(0, RPT // K, _zacc, 0)
            if pp == 0:
                @pl.when(tid == 0)
                def _zdn():
                    pltpu.sync_copy(zro_hbm, dn_sh)
            plsc.subcore_barrier()

            for i in range(4):
                _prep(i, i)
                if i >= 2:
                    _scale_scatter(i - 2, i - 2)

            def _quad(q, carry2):
                for r in range(4):
                    step = 4 + 4 * q + r
                    _wait_sc(r)
                    _prep(step, r)
                    _scale_scatter(step - 2, (r + 2) % 4)
                return carry2

            lax.fori_loop(0, (NCH - 5) // 4, _quad, 0)
            _wait_sc(0)
            _prep(NCH - 1, 0)
            _scale_scatter(NCH - 3, 2)
            _scale_scatter(NCH - 2, 3)
            _scale_scatter(NCH - 1, 0)
            for b in (1, 2, 3, 0):
                _wait_sc(b)
            plsc.subcore_barrier()

            pltpu.sync_copy(acc_sh.at[pl.ds(tid * RPT, RPT)],
                            accs[pp].at[s, pl.ds(tid * RPT, RPT)])
            if pp == 0:
                pltpu.sync_copy(dn_sh.at[pl.ds(tid * RPT, RPT)],
                                dn_hbm.at[s, pl.ds(tid * RPT, RPT)])
            plsc.subcore_barrier()
        return carry

    lax.fori_loop(0, S_PER_CORE, _snapshot, 0)


def _edge(h_half, asv, adv, zro, code):
    mesh = plsc.VectorSubcoreMesh(core_axis_name="c", subcore_axis_name="s")
    fn = functools.partial(
        pl.kernel,
        mesh=mesh,
        compiler_params=pltpu.CompilerParams(needs_layout_passes=False,
                                             use_tc_tiling_on_sc=False),
        out_type=[
            jax.ShapeDtypeStruct((S, Np, DH), f32),
            jax.ShapeDtypeStruct((S, Np, DH), f32),
            jax.ShapeDtypeStruct((S, Np), f32),
        ],
        scratch_types=[
            pltpu.VMEM((Np,), f32),
            pltpu.VMEM((Np,), f32),
            pltpu.VMEM((NCH, K), i32),
            pltpu.VMEM((NCH, K), f32),
            pltpu.VMEM((NR, K), i32),
            pltpu.VMEM((NR, K), i32),
            pltpu.VMEM((NR, K), f32),
            pltpu.VMEM((K, DH), f32),
            pltpu.VMEM((K, DH), f32),
            pltpu.VMEM((K, DH), f32),
            pltpu.VMEM((K, DH), f32),
            pltpu.VMEM_SHARED((Np, DH), f32),
            pltpu.VMEM_SHARED((Np,), f32),
            pltpu.SemaphoreType.DMA,
            pltpu.SemaphoreType.DMA,
            pltpu.SemaphoreType.DMA,
            pltpu.SemaphoreType.DMA,
            pltpu.SemaphoreType.DMA,
            pltpu.SemaphoreType.DMA,
            pltpu.SemaphoreType.DMA,
            pltpu.SemaphoreType.DMA,
            pltpu.SemaphoreType.DMA,
            pltpu.SemaphoreType.DMA,
            pltpu.SemaphoreType.DMA,
            pltpu.SemaphoreType.DMA,
        ],
    )(_edge_body)
    return fn(h_half, asv, adv, zro, code)


def kernel(x, edge_index, valid_step_index, W1, a_src1, a_dst1, b1,
           W2, a_src2, a_dst2, b2, Wt, bt):
    x = x.astype(f32)
    ei = edge_index.astype(i32)
    code = (lax.shift_left(ei[:, :, 0, :], CSH) | ei[:, :, 1, :]).reshape(
        S, NTILES, NCH, K)
    zro = jnp.zeros((Np,), f32)
    xs = x.reshape(S, N, D)

    h1, as1, ad1 = _prep(xs, W1, a_src1.reshape(1, D), a_dst1.reshape(1, D))
    a1h0, a1h1, dn1 = _edge(h1.reshape(S * Np * 2, DH), as1.reshape(S, Np),
                            ad1.reshape(S, Np), zro, code)
    h2, as2, ad2 = _mid(a1h0, a1h1, dn1.reshape(S, Np // 128, 128), h1,
                        b1.reshape(1, D), W2,
                        a_src2.reshape(1, D), a_dst2.reshape(1, D))
    a2h0, a2h1, dn2 = _edge(h2.reshape(S * Np * 2, DH), as2.reshape(S, Np),
                            ad2.reshape(S, Np), zro, code)
    return _final(a2h0.reshape(B, T, Np, DH), a2h1.reshape(B, T, Np, DH),
                  dn2.reshape(B, T, Np // 128, 128),
                  h2.reshape(B, T, Np, D), b2.reshape(1, D),
                  valid_step_index.astype(i32), Wt.astype(f32), bt.astype(f32))
